# Initial kernel scaffold; baseline (speedup 1.0000x reference)
import jax
import jax.numpy as jnp
from jax.experimental import pallas as pl


def kernel(x_nchw, w_oihw, gamma, beta):
    raise NotImplementedError("write your pallas kernel here")


# speedup vs baseline: 1.0702x; 1.0702x over previous
import functools

import jax
import jax.numpy as jnp
from jax.experimental import pallas as pl
from jax.experimental.pallas import tpu as pltpu

_EPS = 1e-5
_SUBLANE = 8


def _conv_stats_kernel(x_ref, w_ref, conv_ref, stats_ref, *,
                       ksize, h_out, w_out):
    x = x_ref[...]
    cin = x.shape[-1]
    taps = []
    for kh in range(ksize):
        for kw in range(ksize):
            t = x[kh:kh + h_out, kw:kw + w_out, :]
            taps.append(t.reshape(h_out * w_out, cin))
    patches = jnp.concatenate(taps, axis=1)
    conv = jax.lax.dot_general(
        w_ref[...], patches,
        dimension_numbers=(((1,), (1,)), ((), ())),
        preferred_element_type=jnp.float32)
    s = jnp.sum(conv, axis=1, keepdims=True)
    sq = jnp.sum(conv * conv, axis=1, keepdims=True)
    stats_ref[...] = jnp.concatenate([s, sq], axis=1)
    conv_ref[...] = conv.astype(jnp.bfloat16)


def _affine_kernel(conv_ref, scale_ref, shift_ref, o_ref):
    o_ref[...] = (conv_ref[...].astype(jnp.float32) * scale_ref[...]
                  + shift_ref[...])


@jax.jit
def _conv_bn(x_nchw, w_oihw, gamma, beta):
    ksize, pad = 3, 1
    n, cin, h, w = x_nchw.shape
    cout = w_oihw.shape[0]
    h_out = h + 2 * pad - ksize + 1
    w_out = w + 2 * pad - ksize + 1
    hw = h_out * w_out
    k_dim = ksize * ksize * cin
    cout_p = -(-cout // _SUBLANE) * _SUBLANE
    hp, wp = h + 2 * pad, w + 2 * pad

    x_pad = jnp.pad(
        jnp.transpose(x_nchw, (0, 2, 3, 1)).astype(jnp.bfloat16),
        ((0, 0), (pad, pad), (pad, pad), (0, 0)))
    w_f = jnp.transpose(w_oihw, (0, 2, 3, 1)).reshape(cout, k_dim)
    w_f = jnp.pad(w_f, ((0, cout_p - cout), (0, 0))).astype(jnp.bfloat16)

    cparams = pltpu.CompilerParams(
        dimension_semantics=("parallel",),
        vmem_limit_bytes=64 * 1024 * 1024,
    )
    x_spec = pl.BlockSpec((None, hp, wp, cin), lambda i: (i, 0, 0, 0))
    w_spec = pl.BlockSpec((cout_p, k_dim), lambda i: (0, 0))

    conv_flat, stats = pl.pallas_call(
        functools.partial(_conv_stats_kernel, ksize=ksize,
                          h_out=h_out, w_out=w_out),
        out_shape=(jax.ShapeDtypeStruct((n, cout_p, hw), jnp.bfloat16),
                   jax.ShapeDtypeStruct((n, cout_p, 2), jnp.float32)),
        grid=(n,),
        in_specs=[x_spec, w_spec],
        out_specs=(pl.BlockSpec((None, cout_p, hw), lambda i: (i, 0, 0)),
                   pl.BlockSpec((None, cout_p, 2), lambda i: (i, 0, 0))),
        compiler_params=cparams,
    )(x_pad, w_f)

    tot = jnp.sum(stats, axis=0)
    count = float(n * hw)
    mean = tot[:, 0] / count
    var = tot[:, 1] / count - mean * mean
    gamma_p = jnp.pad(gamma.astype(jnp.float32), (0, cout_p - cout))
    beta_p = jnp.pad(beta.astype(jnp.float32), (0, cout_p - cout))
    scale = (gamma_p * jax.lax.rsqrt(var + _EPS)).reshape(cout_p, 1)
    shift = (beta_p - mean * scale[:, 0]).reshape(cout_p, 1)

    out_flat = pl.pallas_call(
        _affine_kernel,
        out_shape=jax.ShapeDtypeStruct((n, cout, hw), jnp.float32),
        grid=(n,),
        in_specs=[pl.BlockSpec((None, cout, hw), lambda i: (i, 0, 0)),
                  pl.BlockSpec((cout_p, 1), lambda i: (0, 0)),
                  pl.BlockSpec((cout_p, 1), lambda i: (0, 0))],
        out_specs=pl.BlockSpec((None, cout, hw), lambda i: (i, 0, 0)),
        compiler_params=cparams,
    )(conv_flat[:, :cout, :], scale, shift)

    return out_flat.reshape(n, cout, h_out, w_out)


def kernel(x_nchw, w_oihw, gamma, beta):
    return _conv_bn(x_nchw, w_oihw, gamma, beta)


# speedup vs baseline: 1.1390x; 1.0644x over previous
import functools

import jax
import jax.numpy as jnp
from jax.experimental import pallas as pl
from jax.experimental.pallas import tpu as pltpu

_EPS = 1e-5
_SUBLANE = 8
_IMGS_PER_STEP = 4


def _conv_stats_kernel(x_ref, w_ref, conv_ref, stats_ref, *,
                       ksize, h_out, w_out):
    g = x_ref.shape[0]
    cin = x_ref.shape[-1]
    stot = None
    sqtot = None
    for i in range(g):
        x = x_ref[i]
        taps = []
        for kh in range(ksize):
            for kw in range(ksize):
                t = x[kh:kh + h_out, kw:kw + w_out, :]
                taps.append(t.reshape(h_out * w_out, cin))
        patches = jnp.concatenate(taps, axis=1)
        conv = jax.lax.dot_general(
            w_ref[...], patches,
            dimension_numbers=(((1,), (1,)), ((), ())),
            preferred_element_type=jnp.float32)
        s = jnp.sum(conv, axis=1, keepdims=True)
        sq = jnp.sum(conv * conv, axis=1, keepdims=True)
        stot = s if stot is None else stot + s
        sqtot = sq if sqtot is None else sqtot + sq
        conv_ref[i] = conv.astype(jnp.bfloat16)
    stats_ref[...] = jnp.concatenate([stot, sqtot], axis=1)


def _affine_kernel(conv_ref, scale_ref, shift_ref, o_ref):
    o_ref[...] = (conv_ref[...].astype(jnp.float32) * scale_ref[...][None]
                  + shift_ref[...][None])


@jax.jit
def _conv_bn(x_nchw, w_oihw, gamma, beta):
    ksize, pad = 3, 1
    n, cin, h, w = x_nchw.shape
    cout = w_oihw.shape[0]
    h_out = h + 2 * pad - ksize + 1
    w_out = w + 2 * pad - ksize + 1
    hw = h_out * w_out
    k_dim = ksize * ksize * cin
    cout_p = -(-cout // _SUBLANE) * _SUBLANE
    hp, wp = h + 2 * pad, w + 2 * pad
    grp = _IMGS_PER_STEP if n % _IMGS_PER_STEP == 0 else 1
    n_grp = n // grp

    x_pad = jnp.pad(
        jnp.transpose(x_nchw, (0, 2, 3, 1)).astype(jnp.bfloat16),
        ((0, 0), (pad, pad), (pad, pad), (0, 0)))
    w_f = jnp.transpose(w_oihw, (0, 2, 3, 1)).reshape(cout, k_dim)
    w_f = jnp.pad(w_f, ((0, cout_p - cout), (0, 0))).astype(jnp.bfloat16)

    cparams = pltpu.CompilerParams(
        dimension_semantics=("parallel",),
        vmem_limit_bytes=100 * 1024 * 1024,
    )
    x_spec = pl.BlockSpec((grp, hp, wp, cin), lambda i: (i, 0, 0, 0))
    w_spec = pl.BlockSpec((cout_p, k_dim), lambda i: (0, 0))

    conv_flat, stats = pl.pallas_call(
        functools.partial(_conv_stats_kernel, ksize=ksize,
                          h_out=h_out, w_out=w_out),
        out_shape=(jax.ShapeDtypeStruct((n, cout_p, hw), jnp.bfloat16),
                   jax.ShapeDtypeStruct((n_grp, cout_p, 2), jnp.float32)),
        grid=(n_grp,),
        in_specs=[x_spec, w_spec],
        out_specs=(pl.BlockSpec((grp, cout_p, hw), lambda i: (i, 0, 0)),
                   pl.BlockSpec((None, cout_p, 2), lambda i: (i, 0, 0))),
        compiler_params=cparams,
    )(x_pad, w_f)

    tot = jnp.sum(stats, axis=0)
    count = float(n * hw)
    mean = tot[:, 0] / count
    var = tot[:, 1] / count - mean * mean
    gamma_p = jnp.pad(gamma.astype(jnp.float32), (0, cout_p - cout))
    beta_p = jnp.pad(beta.astype(jnp.float32), (0, cout_p - cout))
    scale = (gamma_p * jax.lax.rsqrt(var + _EPS)).reshape(cout_p, 1)
    shift = (beta_p - mean * scale[:, 0]).reshape(cout_p, 1)

    out_flat = pl.pallas_call(
        _affine_kernel,
        out_shape=jax.ShapeDtypeStruct((n, cout, hw), jnp.float32),
        grid=(n_grp,),
        in_specs=[pl.BlockSpec((grp, cout, hw), lambda i: (i, 0, 0)),
                  pl.BlockSpec((cout_p, 1), lambda i: (0, 0)),
                  pl.BlockSpec((cout_p, 1), lambda i: (0, 0))],
        out_specs=pl.BlockSpec((grp, cout, hw), lambda i: (i, 0, 0)),
        compiler_params=cparams,
    )(conv_flat[:, :cout, :], scale, shift)

    return out_flat.reshape(n, cout, h_out, w_out)


def kernel(x_nchw, w_oihw, gamma, beta):
    return _conv_bn(x_nchw, w_oihw, gamma, beta)
